# baseline (speedup 1.0000x reference)
import functools

import jax
import jax.numpy as jnp
from jax import lax
from jax.experimental import pallas as pl
from jax.experimental.pallas import tpu as pltpu
from jax.experimental.pallas import tpu_sc as plsc

N0, N1, N2, C = 10000, 20000, 10000, 512
OB = 2 * N0
OC = 2 * N0 + 3 * N1
TROWS = OC + 2 * N2
LG = C // 16

NNZ_PAD = 460800
NTILES = 16
SLICE = NNZ_PAD // NTILES
W = 3200
NWIN = SLICE // W
CH = 2048
NCHUNK = 20
DPAD = NCHUNK * CH
CPS = NCHUNK // 2
OWN = CH // NTILES
B = 64
SCAP = NTILES * W + 256
SENT = jnp.int32(1 << 20)


def _mm_body(x_ref, w_ref, o_ref):
    o_ref[...] = jnp.dot(x_ref[...], w_ref[...],
                         preferred_element_type=jnp.float32)


def _mm(x, w, bm=400):
    n, c = x.shape
    k = w.shape[1]
    return pl.pallas_call(
        _mm_body,
        grid=(n // bm,),
        in_specs=[
            pl.BlockSpec((bm, c), lambda i: (i, 0)),
            pl.BlockSpec((c, k), lambda i: (0, 0)),
        ],
        out_specs=pl.BlockSpec((bm, k), lambda i: (i, 0)),
        out_shape=jax.ShapeDtypeStruct((n, k), jnp.float32),
    )(x, w)


def _sc_body(t_hbm, dst_hbm, src_hbm, val_hbm, out_hbm,
             dwin, swin, vwin, pd, ps, pv, cbuf, crd, sd, ss, sv,
             qd, qs, qv, gi, gbuf, acc,
             sl_d, sl_s, sl_v, scnt, gsem):
    c = lax.axis_index("c")
    s = lax.axis_index("s")

    def _zrow(r, carry):
        base = r * C
        for jj in range(LG):
            acc[pl.ds(base + jj * 16, 16)] = jnp.zeros((16,), jnp.float32)
        return carry

    lax.fori_loop(0, OWN, _zrow, 0)

    def qflush():
        for g in range(B // 16):
            gi[pl.ds(g * 16, 16)] = qs[pl.ds(g * 16, 16)]
        pltpu.async_copy(t_hbm.at[gi], gbuf, gsem).wait()
        for g in range(B // 16):

            def _row(r, carry, g=g):
                vg = qv[pl.ds(g * 16, 16)]
                dgm = (qd[pl.ds(g * 16, 16)] & (OWN - 1)) * C
                rf = jnp.full((16,), r, jnp.int32)
                bcv = vg.at[rf].get(mode="promise_in_bounds")
                av = dgm.at[rf].get(mode="promise_in_bounds")
                abase = av[0]
                row = g * 16 + r
                for jj in range(LG):
                    plsc.addupdate(
                        acc.at[pl.ds(abase + jj * 16, 16)],
                        gbuf[row, pl.ds(jj * 16, 16)] * bcv)
                return carry

            lax.fori_loop(0, 16, _row, 0)

    def chunk_body(j, carry):
        lo = (2 * j + c) * CH

        def round_body(w, pend):
            base = s * SLICE + w * W
            pltpu.sync_copy(dst_hbm.at[pl.ds(base, W)], dwin)
            pltpu.sync_copy(src_hbm.at[pl.ds(base, W)], swin)
            pltpu.sync_copy(val_hbm.at[pl.ds(base, W)], vwin)

            def grp(i, cnt):
                d = dwin[pl.ds(i * 16, 16)]
                dl = d - lo
                m = (dl >= 0) & (dl < CH)
                nm = plsc.all_reduce_population_count(m)[0]

                @pl.when(nm > 0)
                def _():
                    sv_ = swin[pl.ds(i * 16, 16)]
                    vv = vwin[pl.ds(i * 16, 16)]
                    plsc.store_compressed(pd.at[pl.ds(cnt, 16)], dl, mask=m)
                    plsc.store_compressed(ps.at[pl.ds(cnt, 16)], sv_, mask=m)
                    plsc.store_compressed(pv.at[pl.ds(cnt, 16)], vv, mask=m)

                return cnt + nm

            cnt = lax.fori_loop(0, W // 16, grp, jnp.int32(0))
            pd[pl.ds(cnt, 16)] = jnp.full((16,), SENT, jnp.int32)
            cnt16 = (cnt + 15) & ~15
            cbuf[pl.ds(0, 16)] = jnp.full((16,), cnt16, jnp.int32)
            pltpu.sync_copy(cbuf.at[pl.ds(0, 8)], scnt.at[pl.ds(s * 8, 8)])
            plsc.subcore_barrier()

            pltpu.sync_copy(scnt, crd)
            lanes = lax.iota(jnp.int32, 16)
            counts = jnp.zeros((16,), jnp.int32)
            for g in range(8):
                cg = crd[pl.ds(g * 16, 16)]
                c0 = cg.at[jnp.full((16,), 0, jnp.int32)].get(
                    mode="promise_in_bounds")
                c1 = cg.at[jnp.full((16,), 8, jnp.int32)].get(
                    mode="promise_in_bounds")
                counts = counts + jnp.where(lanes == 2 * g, c0, 0)
                counts = counts + jnp.where(lanes == 2 * g + 1, c1, 0)
            mybase = pl.multiple_of(jnp.sum(jnp.where(lanes < s, counts, 0)),
                                    16)
            total = jnp.sum(counts)

            def pub64(g, carry2):
                o = g * 64
                pltpu.sync_copy(pd.at[pl.ds(o, 64)],
                                sl_d.at[pl.ds(mybase + o, 64)])
                pltpu.sync_copy(ps.at[pl.ds(o, 64)],
                                sl_s.at[pl.ds(mybase + o, 64)])
                pltpu.sync_copy(pv.at[pl.ds(o, 64)],
                                sl_v.at[pl.ds(mybase + o, 64)])
                return carry2

            lax.fori_loop(0, cnt16 // 64, pub64, 0)

            def pub16(g, carry2):
                o = g * 16
                pltpu.sync_copy(pd.at[pl.ds(o, 16)],
                                sl_d.at[pl.ds(mybase + o, 16)])
                pltpu.sync_copy(ps.at[pl.ds(o, 16)],
                                sl_s.at[pl.ds(mybase + o, 16)])
                pltpu.sync_copy(pv.at[pl.ds(o, 16)],
                                sl_v.at[pl.ds(mybase + o, 16)])
                return carry2

            lax.fori_loop((cnt16 // 64) * 4, cnt16 // 16, pub16, 0)
            plsc.subcore_barrier()

            nblk = (total + 255) // 256

            def sblk(g2, pend):
                pltpu.sync_copy(sl_d.at[pl.ds(g2 * 256, 256)], sd)
                pltpu.sync_copy(sl_s.at[pl.ds(g2 * 256, 256)], ss)
                pltpu.sync_copy(sl_v.at[pl.ds(g2 * 256, 256)], sv)

                def cgrp(i2, pend2):
                    lanes2 = lax.iota(jnp.int32, 16)
                    gidx = g2 * 256 + i2 * 16 + lanes2
                    dl = sd[pl.ds(i2 * 16, 16)]
                    mine = (gidx < total) & ((dl >> 7) == s) & (dl >= 0) \
                        & (dl < CH)
                    nm = plsc.all_reduce_population_count(mine)[0]

                    @pl.when(nm > 0)
                    def _():
                        sv2 = ss[pl.ds(i2 * 16, 16)]
                        vv2 = sv[pl.ds(i2 * 16, 16)]
                        plsc.store_compressed(qd.at[pl.ds(pend2, 16)], dl,
                                              mask=mine)
                        plsc.store_compressed(qs.at[pl.ds(pend2, 16)], sv2,
                                              mask=mine)
                        plsc.store_compressed(qv.at[pl.ds(pend2, 16)], vv2,
                                              mask=mine)

                    pend3 = pend2 + nm

                    @pl.when(pend3 >= B)
                    def _():
                        qflush()
                        lanes3 = lax.iota(jnp.int32, 16)
                        keep = lanes3 < (pend3 - B)
                        tl_d = qd[pl.ds(B, 16)]
                        tl_s = qs[pl.ds(B, 16)]
                        tl_v = qv[pl.ds(B, 16)]
                        qd[pl.ds(0, 16)] = jnp.where(keep, tl_d,
                                                     qd[pl.ds(0, 16)])
                        qs[pl.ds(0, 16)] = jnp.where(keep, tl_s,
                                                     qs[pl.ds(0, 16)])
                        qv[pl.ds(0, 16)] = jnp.where(keep, tl_v,
                                                     qv[pl.ds(0, 16)])

                    return jnp.where(pend3 >= B, pend3 - B, pend3)

                return lax.fori_loop(0, 16, cgrp, pend)

            return lax.fori_loop(0, nblk, sblk, pend)

        pend = lax.fori_loop(0, NWIN, round_body, jnp.int32(0))

        lanes = lax.iota(jnp.int32, 16)
        for g in range(B // 16):
            idx = lanes + g * 16
            mpad = idx >= pend
            qd[pl.ds(g * 16, 16)] = jnp.where(mpad, s * OWN,
                                              qd[pl.ds(g * 16, 16)])
            qs[pl.ds(g * 16, 16)] = jnp.where(mpad, s * 16 + lanes,
                                              qs[pl.ds(g * 16, 16)])
            qv[pl.ds(g * 16, 16)] = jnp.where(mpad, 0.0,
                                              qv[pl.ds(g * 16, 16)])
        qflush()

        def _sig(r, carry2):
            base = r * C
            for jj in range(LG):
                x = acc[pl.ds(base + jj * 16, 16)]
                acc[pl.ds(base + jj * 16, 16)] = 1.0 / (1.0 + jnp.exp(-x))
            return carry2

        lax.fori_loop(0, OWN, _sig, 0)
        obase = pl.multiple_of((lo + s * OWN) * C, OWN * C)
        pltpu.sync_copy(acc, out_hbm.at[pl.ds(obase, OWN * C)])
        lax.fori_loop(0, OWN, _zrow, 0)
        return carry

    lax.fori_loop(0, CPS, chunk_body, 0)


@jax.jit
def _sc_spmm(t, dst, src, val):
    mesh = plsc.VectorSubcoreMesh(core_axis_name="c", subcore_axis_name="s")
    f = pl.kernel(
        _sc_body,
        out_type=jax.ShapeDtypeStruct((DPAD * C,), jnp.float32),
        mesh=mesh,
        compiler_params=pltpu.CompilerParams(needs_layout_passes=False),
        scratch_types=[
            pltpu.VMEM((W,), jnp.int32),
            pltpu.VMEM((W,), jnp.int32),
            pltpu.VMEM((W,), jnp.float32),
            pltpu.VMEM((W + 16,), jnp.int32),
            pltpu.VMEM((W + 16,), jnp.int32),
            pltpu.VMEM((W + 16,), jnp.float32),
            pltpu.VMEM((16,), jnp.int32),
            pltpu.VMEM((128,), jnp.int32),
            pltpu.VMEM((256,), jnp.int32),
            pltpu.VMEM((256,), jnp.int32),
            pltpu.VMEM((256,), jnp.float32),
            pltpu.VMEM((B + 16,), jnp.int32),
            pltpu.VMEM((B + 16,), jnp.int32),
            pltpu.VMEM((B + 16,), jnp.float32),
            pltpu.VMEM((B,), jnp.int32),
            pltpu.VMEM((B, C), jnp.float32),
            pltpu.VMEM((OWN * C,), jnp.float32),
            pltpu.VMEM_SHARED((SCAP,), jnp.int32),
            pltpu.VMEM_SHARED((SCAP,), jnp.int32),
            pltpu.VMEM_SHARED((SCAP,), jnp.float32),
            pltpu.VMEM_SHARED((128,), jnp.int32),
            pltpu.SemaphoreType.DMA,
        ],
    )
    return f(t, dst, src, val)


def kernel(x0, x1, x2, adj0_idx, adj0_val, adj1_idx, adj1_val, adj2_idx,
           adj2_val, inc1_rows, inc1_cols, inc1_val, inc2_rows, inc2_cols,
           inc2_val, W_same_0, W_same_1, W_same_2, W_l2h_1, W_l2h_2,
           W_h2l_0, W_h2l_1):
    ya = _mm(x0, jnp.concatenate([W_same_0, W_l2h_1], axis=1))
    yb = _mm(x1, jnp.concatenate([W_same_1, W_h2l_0, W_l2h_2], axis=1))
    yc = _mm(x2, jnp.concatenate([W_same_2, W_h2l_1], axis=1))
    t = jnp.concatenate([ya.reshape(2 * N0, C), yb.reshape(3 * N1, C),
                         yc.reshape(2 * N2, C)], axis=0)

    i32 = jnp.int32
    dst = jnp.concatenate([
        adj0_idx[0], inc1_rows, N0 + adj1_idx[0], N0 + inc2_rows,
        N0 + inc1_cols, N0 + N1 + adj2_idx[0], N0 + N1 + inc2_cols,
    ]).astype(i32)
    src = jnp.concatenate([
        2 * adj0_idx[1], OB + 3 * inc1_cols + 1, OB + 3 * adj1_idx[1],
        OC + 2 * inc2_cols + 1, 2 * inc1_rows + 1, OC + 2 * adj2_idx[1],
        OB + 3 * inc2_rows + 2,
    ]).astype(i32)
    val = jnp.concatenate([
        adj0_val, inc1_val, adj1_val, inc2_val, inc1_val, adj2_val, inc2_val,
    ])
    pad = NNZ_PAD - dst.shape[0]
    dst = jnp.concatenate([dst, jnp.zeros((pad,), i32)])
    src = jnp.concatenate([src, jnp.arange(pad, dtype=i32) % 256])
    val = jnp.concatenate([val, jnp.zeros((pad,), jnp.float32)])

    out = _sc_spmm(t, dst, src, val).reshape(DPAD, C)
    return (out[:N0], out[N0:N0 + N1], out[N0 + N1:N0 + N1 + N2])


# speedup vs baseline: 1.0326x; 1.0326x over previous
import functools

import jax
import jax.numpy as jnp
from jax import lax
from jax.experimental import pallas as pl
from jax.experimental.pallas import tpu as pltpu
from jax.experimental.pallas import tpu_sc as plsc

N0, N1, N2, C = 10000, 20000, 10000, 512
OB = 2 * N0
OC = 2 * N0 + 3 * N1
TROWS = OC + 2 * N2
LG = C // 16

NNZ_PAD = 460800
NTILES = 16
SLICE = NNZ_PAD // NTILES
W = 3200
NWIN = SLICE // W
CH = 2048
NCHUNK = 20
DPAD = NCHUNK * CH
CPS = NCHUNK // 2
OWN = CH // NTILES
B = 64
SCAP = NTILES * W + 256
SENT = jnp.int32(1 << 20)


def _mm_body(x_ref, w_ref, o_ref):
    o_ref[...] = jnp.dot(x_ref[...], w_ref[...],
                         preferred_element_type=jnp.float32)


def _mm(x, w, bm=400):
    n, c = x.shape
    k = w.shape[1]
    return pl.pallas_call(
        _mm_body,
        grid=(n // bm,),
        in_specs=[
            pl.BlockSpec((bm, c), lambda i: (i, 0)),
            pl.BlockSpec((c, k), lambda i: (0, 0)),
        ],
        out_specs=pl.BlockSpec((bm, k), lambda i: (i, 0)),
        out_shape=jax.ShapeDtypeStruct((n, k), jnp.float32),
    )(x, w)


def _sc_body(t_hbm, dst_hbm, src_hbm, val_hbm, out_hbm,
             dwin, swin, vwin, pd, ps, pv, cbuf, crd, sd, ss, sv,
             qd, qs, qv, gi, gbuf, acc,
             sl_d, sl_s, sl_v, scnt, gsem):
    c = lax.axis_index("c")
    s = lax.axis_index("s")

    def _zrow(r, carry):
        base = r * C
        for jj in range(LG):
            acc[pl.ds(base + jj * 16, 16)] = jnp.zeros((16,), jnp.float32)
        return carry

    lax.fori_loop(0, OWN, _zrow, 0)

    def qflush():
        for g in range(B // 16):
            gi[pl.ds(g * 16, 16)] = qs[pl.ds(g * 16, 16)]
        pltpu.async_copy(t_hbm.at[gi], gbuf, gsem).wait()
        for g in range(B // 16):

            def _row(r, carry, g=g):
                lanes = lax.iota(jnp.int32, 16)
                vg = qv[pl.ds(g * 16, 16)]
                dgm = (qd[pl.ds(g * 16, 16)] & (OWN - 1)) * C
                rf = jnp.full((16,), r, jnp.int32)
                rf2 = rf + 8
                bcv = vg.at[rf].get(mode="promise_in_bounds")
                av = dgm.at[rf].get(mode="promise_in_bounds")
                bcv2 = vg.at[rf2].get(mode="promise_in_bounds")
                av2 = dgm.at[rf2].get(mode="promise_in_bounds")
                row = g * 16 + r
                row2 = row + 8
                for jj in range(LG):
                    col = (jj * 16) + lanes
                    plsc.addupdate_scatter(
                        acc, [av + col],
                        gbuf[row, pl.ds(jj * 16, 16)] * bcv)
                    plsc.addupdate_scatter(
                        acc, [av2 + col],
                        gbuf[row2, pl.ds(jj * 16, 16)] * bcv2)
                return carry

            lax.fori_loop(0, 8, _row, 0)

    def chunk_body(j, carry):
        lo = (2 * j + c) * CH

        def round_body(w, pend):
            base = s * SLICE + w * W
            pltpu.sync_copy(dst_hbm.at[pl.ds(base, W)], dwin)
            pltpu.sync_copy(src_hbm.at[pl.ds(base, W)], swin)
            pltpu.sync_copy(val_hbm.at[pl.ds(base, W)], vwin)

            def grp(i, cnt):
                d = dwin[pl.ds(i * 16, 16)]
                dl = d - lo
                m = (dl >= 0) & (dl < CH)
                nm = plsc.all_reduce_population_count(m)[0]

                @pl.when(nm > 0)
                def _():
                    sv_ = swin[pl.ds(i * 16, 16)]
                    vv = vwin[pl.ds(i * 16, 16)]
                    plsc.store_compressed(pd.at[pl.ds(cnt, 16)], dl, mask=m)
                    plsc.store_compressed(ps.at[pl.ds(cnt, 16)], sv_, mask=m)
                    plsc.store_compressed(pv.at[pl.ds(cnt, 16)], vv, mask=m)

                return cnt + nm

            cnt = lax.fori_loop(0, W // 16, grp, jnp.int32(0))
            pd[pl.ds(cnt, 16)] = jnp.full((16,), SENT, jnp.int32)
            cnt16 = (cnt + 15) & ~15
            cbuf[pl.ds(0, 16)] = jnp.full((16,), cnt16, jnp.int32)
            pltpu.sync_copy(cbuf.at[pl.ds(0, 8)], scnt.at[pl.ds(s * 8, 8)])
            plsc.subcore_barrier()

            pltpu.sync_copy(scnt, crd)
            lanes = lax.iota(jnp.int32, 16)
            counts = jnp.zeros((16,), jnp.int32)
            for g in range(8):
                cg = crd[pl.ds(g * 16, 16)]
                c0 = cg.at[jnp.full((16,), 0, jnp.int32)].get(
                    mode="promise_in_bounds")
                c1 = cg.at[jnp.full((16,), 8, jnp.int32)].get(
                    mode="promise_in_bounds")
                counts = counts + jnp.where(lanes == 2 * g, c0, 0)
                counts = counts + jnp.where(lanes == 2 * g + 1, c1, 0)
            mybase = pl.multiple_of(jnp.sum(jnp.where(lanes < s, counts, 0)),
                                    16)
            total = jnp.sum(counts)

            def pub64(g, carry2):
                o = g * 64
                pltpu.sync_copy(pd.at[pl.ds(o, 64)],
                                sl_d.at[pl.ds(mybase + o, 64)])
                pltpu.sync_copy(ps.at[pl.ds(o, 64)],
                                sl_s.at[pl.ds(mybase + o, 64)])
                pltpu.sync_copy(pv.at[pl.ds(o, 64)],
                                sl_v.at[pl.ds(mybase + o, 64)])
                return carry2

            lax.fori_loop(0, cnt16 // 64, pub64, 0)

            def pub16(g, carry2):
                o = g * 16
                pltpu.sync_copy(pd.at[pl.ds(o, 16)],
                                sl_d.at[pl.ds(mybase + o, 16)])
                pltpu.sync_copy(ps.at[pl.ds(o, 16)],
                                sl_s.at[pl.ds(mybase + o, 16)])
                pltpu.sync_copy(pv.at[pl.ds(o, 16)],
                                sl_v.at[pl.ds(mybase + o, 16)])
                return carry2

            lax.fori_loop((cnt16 // 64) * 4, cnt16 // 16, pub16, 0)
            plsc.subcore_barrier()

            nblk = (total + 255) // 256

            def sblk(g2, pend):
                pltpu.sync_copy(sl_d.at[pl.ds(g2 * 256, 256)], sd)
                pltpu.sync_copy(sl_s.at[pl.ds(g2 * 256, 256)], ss)
                pltpu.sync_copy(sl_v.at[pl.ds(g2 * 256, 256)], sv)

                def cgrp(i2, pend2):
                    lanes2 = lax.iota(jnp.int32, 16)
                    gidx = g2 * 256 + i2 * 16 + lanes2
                    dl = sd[pl.ds(i2 * 16, 16)]
                    mine = (gidx < total) & ((dl >> 7) == s) & (dl >= 0) \
                        & (dl < CH)
                    nm = plsc.all_reduce_population_count(mine)[0]

                    @pl.when(nm > 0)
                    def _():
                        sv2 = ss[pl.ds(i2 * 16, 16)]
                        vv2 = sv[pl.ds(i2 * 16, 16)]
                        plsc.store_compressed(qd.at[pl.ds(pend2, 16)], dl,
                                              mask=mine)
                        plsc.store_compressed(qs.at[pl.ds(pend2, 16)], sv2,
                                              mask=mine)
                        plsc.store_compressed(qv.at[pl.ds(pend2, 16)], vv2,
                                              mask=mine)

                    pend3 = pend2 + nm

                    @pl.when(pend3 >= B)
                    def _():
                        qflush()
                        lanes3 = lax.iota(jnp.int32, 16)
                        keep = lanes3 < (pend3 - B)
                        tl_d = qd[pl.ds(B, 16)]
                        tl_s = qs[pl.ds(B, 16)]
                        tl_v = qv[pl.ds(B, 16)]
                        qd[pl.ds(0, 16)] = jnp.where(keep, tl_d,
                                                     qd[pl.ds(0, 16)])
                        qs[pl.ds(0, 16)] = jnp.where(keep, tl_s,
                                                     qs[pl.ds(0, 16)])
                        qv[pl.ds(0, 16)] = jnp.where(keep, tl_v,
                                                     qv[pl.ds(0, 16)])

                    return jnp.where(pend3 >= B, pend3 - B, pend3)

                return lax.fori_loop(0, 16, cgrp, pend)

            return lax.fori_loop(0, nblk, sblk, pend)

        pend = lax.fori_loop(0, NWIN, round_body, jnp.int32(0))

        lanes = lax.iota(jnp.int32, 16)
        for g in range(B // 16):
            idx = lanes + g * 16
            mpad = idx >= pend
            qd[pl.ds(g * 16, 16)] = jnp.where(mpad, s * OWN,
                                              qd[pl.ds(g * 16, 16)])
            qs[pl.ds(g * 16, 16)] = jnp.where(mpad, s * 16 + lanes,
                                              qs[pl.ds(g * 16, 16)])
            qv[pl.ds(g * 16, 16)] = jnp.where(mpad, 0.0,
                                              qv[pl.ds(g * 16, 16)])
        qflush()

        def _sig(r, carry2):
            base = r * C
            for jj in range(LG):
                x = acc[pl.ds(base + jj * 16, 16)]
                acc[pl.ds(base + jj * 16, 16)] = 1.0 / (1.0 + jnp.exp(-x))
            return carry2

        lax.fori_loop(0, OWN, _sig, 0)
        obase = pl.multiple_of((lo + s * OWN) * C, OWN * C)
        pltpu.sync_copy(acc, out_hbm.at[pl.ds(obase, OWN * C)])
        lax.fori_loop(0, OWN, _zrow, 0)
        return carry

    lax.fori_loop(0, CPS, chunk_body, 0)


@jax.jit
def _sc_spmm(t, dst, src, val):
    mesh = plsc.VectorSubcoreMesh(core_axis_name="c", subcore_axis_name="s")
    f = pl.kernel(
        _sc_body,
        out_type=jax.ShapeDtypeStruct((DPAD * C,), jnp.float32),
        mesh=mesh,
        compiler_params=pltpu.CompilerParams(needs_layout_passes=False),
        scratch_types=[
            pltpu.VMEM((W,), jnp.int32),
            pltpu.VMEM((W,), jnp.int32),
            pltpu.VMEM((W,), jnp.float32),
            pltpu.VMEM((W + 16,), jnp.int32),
            pltpu.VMEM((W + 16,), jnp.int32),
            pltpu.VMEM((W + 16,), jnp.float32),
            pltpu.VMEM((16,), jnp.int32),
            pltpu.VMEM((128,), jnp.int32),
            pltpu.VMEM((256,), jnp.int32),
            pltpu.VMEM((256,), jnp.int32),
            pltpu.VMEM((256,), jnp.float32),
            pltpu.VMEM((B + 16,), jnp.int32),
            pltpu.VMEM((B + 16,), jnp.int32),
            pltpu.VMEM((B + 16,), jnp.float32),
            pltpu.VMEM((B,), jnp.int32),
            pltpu.VMEM((B, C), jnp.float32),
            pltpu.VMEM((OWN * C,), jnp.float32),
            pltpu.VMEM_SHARED((SCAP,), jnp.int32),
            pltpu.VMEM_SHARED((SCAP,), jnp.int32),
            pltpu.VMEM_SHARED((SCAP,), jnp.float32),
            pltpu.VMEM_SHARED((128,), jnp.int32),
            pltpu.SemaphoreType.DMA,
        ],
    )
    return f(t, dst, src, val)


def kernel(x0, x1, x2, adj0_idx, adj0_val, adj1_idx, adj1_val, adj2_idx,
           adj2_val, inc1_rows, inc1_cols, inc1_val, inc2_rows, inc2_cols,
           inc2_val, W_same_0, W_same_1, W_same_2, W_l2h_1, W_l2h_2,
           W_h2l_0, W_h2l_1):
    ya = _mm(x0, jnp.concatenate([W_same_0, W_l2h_1], axis=1))
    yb = _mm(x1, jnp.concatenate([W_same_1, W_h2l_0, W_l2h_2], axis=1))
    yc = _mm(x2, jnp.concatenate([W_same_2, W_h2l_1], axis=1))
    t = jnp.concatenate([ya.reshape(2 * N0, C), yb.reshape(3 * N1, C),
                         yc.reshape(2 * N2, C)], axis=0)

    i32 = jnp.int32
    dst = jnp.concatenate([
        adj0_idx[0], inc1_rows, N0 + adj1_idx[0], N0 + inc2_rows,
        N0 + inc1_cols, N0 + N1 + adj2_idx[0], N0 + N1 + inc2_cols,
    ]).astype(i32)
    src = jnp.concatenate([
        2 * adj0_idx[1], OB + 3 * inc1_cols + 1, OB + 3 * adj1_idx[1],
        OC + 2 * inc2_cols + 1, 2 * inc1_rows + 1, OC + 2 * adj2_idx[1],
        OB + 3 * inc2_rows + 2,
    ]).astype(i32)
    val = jnp.concatenate([
        adj0_val, inc1_val, adj1_val, inc2_val, inc1_val, adj2_val, inc2_val,
    ])
    pad = NNZ_PAD - dst.shape[0]
    dst = jnp.concatenate([dst, jnp.zeros((pad,), i32)])
    src = jnp.concatenate([src, jnp.arange(pad, dtype=i32) % 256])
    val = jnp.concatenate([val, jnp.zeros((pad,), jnp.float32)])

    out = _sc_spmm(t, dst, src, val).reshape(DPAD, C)
    return (out[:N0], out[N0:N0 + N1], out[N0 + N1:N0 + N1 + N2])
